# baseline (speedup 1.0000x reference)
import functools

import jax
import jax.numpy as jnp
from jax import lax
from jax.experimental import pallas as pl
from jax.experimental.pallas import tpu as pltpu
from jax.experimental.pallas import tpu_sc as plsc

N = 10000
E = 160000
D = 128
H = 8
C = 128
L = 4
R = 64
NEG = 0.2
HC = H * C

NP = 10240
NBLK = 1024
NC, NS = 2, 16
NW = NC * NS
NG = E // 16
GPW = (NG + NW - 1) // NW
RPS = NP // NS

_mesh = plsc.VectorSubcoreMesh(core_axis_name="c", subcore_axis_name="s")


def _wid():
    return lax.axis_index("s") * NC + lax.axis_index("c")


def _lanes():
    return lax.iota(jnp.int32, 16)


def _splat(v):
    return jnp.full((16,), v, jnp.int32)


@functools.partial(
    pl.kernel,
    out_type=jax.ShapeDtypeStruct((2 * NP, C), jnp.float32),
    mesh=_mesh,
    compiler_params=pltpu.CompilerParams(needs_layout_passes=False),
    scratch_types=[
        pltpu.VMEM_SHARED((NP, C), jnp.float32),
        pltpu.VMEM((16, C), jnp.float32),
        pltpu.VMEM((16,), jnp.int32),
        pltpu.VMEM((16,), jnp.int32),
    ],
)
def _k0_hist(dst_hbm, rel_hbm, zrows_hbm, hist_out, hist_sh, ones_v, dst_v, rel_v):
    cid = lax.axis_index("c")
    sid = lax.axis_index("s")
    wid = _wid()
    pltpu.sync_copy(zrows_hbm, hist_sh.at[pl.ds(sid * RPS, RPS)])
    plsc.subcore_barrier()
    lanes = _lanes()

    def body(i, carry):
        g = i * NW + wid

        @pl.when(g < NG)
        def _():
            pltpu.sync_copy(dst_hbm.at[pl.ds(g * 16, 16)], dst_v)
            pltpu.sync_copy(rel_hbm.at[pl.ds(g * 16, 16)], rel_v)
            for e in range(16):
                rel_e = plsc.load_gather(rel_v, [_splat(e)])
                for j in range(C // 16):
                    ones_v[e, pl.ds(j * 16, 16)] = jnp.where(
                        j * 16 + lanes == rel_e, 1.0, 0.0
                    )
            pltpu.sync_copy(ones_v, hist_sh.at[dst_v], add=True)

        return carry

    lax.fori_loop(0, GPW, body, 0)
    plsc.subcore_barrier()
    pltpu.sync_copy(
        hist_sh.at[pl.ds(sid * RPS, RPS)],
        hist_out.at[pl.ds(cid * NP + sid * RPS, RPS)],
    )


@functools.partial(
    pl.kernel,
    out_type=(
        jax.ShapeDtypeStruct((NG, 8, 16), jnp.float32),
        jax.ShapeDtypeStruct((NW, 16), jnp.float32),
    ),
    mesh=_mesh,
    compiler_params=pltpu.CompilerParams(needs_layout_passes=False),
    scratch_types=[
        pltpu.VMEM((R, HC), jnp.float32),
        pltpu.VMEM((HC,), jnp.float32),
        pltpu.VMEM((16, HC), jnp.float32),
        pltpu.VMEM((16, HC), jnp.float32),
        pltpu.VMEM((16,), jnp.int32),
        pltpu.VMEM((16,), jnp.int32),
        pltpu.VMEM((16,), jnp.int32),
        pltpu.VMEM((8, 16), jnp.float32),
        pltpu.VMEM((16,), jnp.float32),
        pltpu.SemaphoreType.DMA,
        pltpu.SemaphoreType.DMA,
    ],
)
def _k1_logits(xl_hbm, xr_hbm, re2_hbm, att_hbm, src_hbm, dst_hbm, rel_hbm,
               logits_out, maxw_out,
               re2_v, att_v, xlr, xrr, src_v, dst_v, rel_v, lg_v, mxc_v,
               sem1, sem2):
    wid = _wid()
    pltpu.sync_copy(re2_hbm, re2_v)
    pltpu.sync_copy(att_hbm, att_v)
    lanes = _lanes()
    mxc_v[...] = jnp.full((16,), -1e30, jnp.float32)

    def body(i, carry):
        g = i * NW + wid

        @pl.when(g < NG)
        def _():
            base = g * 16
            pltpu.sync_copy(src_hbm.at[pl.ds(base, 16)], src_v)
            pltpu.sync_copy(dst_hbm.at[pl.ds(base, 16)], dst_v)
            pltpu.sync_copy(rel_hbm.at[pl.ds(base, 16)], rel_v)
            cp1 = pltpu.async_copy(xl_hbm.at[src_v], xlr, sem1)
            cp2 = pltpu.async_copy(xr_hbm.at[dst_v], xrr, sem2)
            cp1.wait()
            cp2.wait()
            relv = rel_v[...]
            mxc = mxc_v[...]
            for h in range(H):
                def fbody(c, acc):
                    f = h * C + c
                    fsp = _splat(f)
                    xlv = plsc.load_gather(xlr, [lanes, fsp])
                    xrv = plsc.load_gather(xrr, [lanes, fsp])
                    rev = plsc.load_gather(re2_v, [relv, fsp])
                    attv = plsc.load_gather(att_v, [fsp])
                    z = xlv + xrv + rev
                    z = jnp.maximum(z, NEG * z)
                    return acc + z * attv

                logit = lax.fori_loop(0, C, fbody, jnp.zeros((16,), jnp.float32))
                lg_v[h] = logit
                mh = jnp.max(logit)
                mxc = jnp.where(lanes == h, jnp.maximum(mxc, mh), mxc)
            mxc_v[...] = mxc
            pltpu.sync_copy(lg_v, logits_out.at[g])

        return carry

    lax.fori_loop(0, GPW, body, 0)
    pltpu.sync_copy(mxc_v, maxw_out.at[wid])


@functools.partial(
    pl.kernel,
    out_type=(
        jax.ShapeDtypeStruct((NG, 8, 16), jnp.float32),
        jax.ShapeDtypeStruct((2 * NP, C), jnp.float32),
    ),
    mesh=_mesh,
    compiler_params=pltpu.CompilerParams(needs_layout_passes=False),
    scratch_types=[
        pltpu.VMEM_SHARED((NP, C), jnp.float32),
        pltpu.VMEM((16,), jnp.float32),
        pltpu.VMEM((8, 16), jnp.float32),
        pltpu.VMEM((8, 16), jnp.float32),
        pltpu.VMEM((16, C), jnp.float32),
        pltpu.VMEM((16,), jnp.int32),
    ],
)
def _k2_softmax(logits_hbm, m_hbm, dst_hbm, zrows_hbm, ex_out, den_out,
                den_sh, m_v, lg_v, ex_v, dr_v, dst_v):
    cid = lax.axis_index("c")
    sid = lax.axis_index("s")
    wid = _wid()
    pltpu.sync_copy(m_hbm, m_v)
    pltpu.sync_copy(zrows_hbm, den_sh.at[pl.ds(sid * RPS, RPS)])
    plsc.subcore_barrier()
    lanes = _lanes()
    hsel = jnp.minimum(lanes, H - 1)
    hmask = lanes < H
    zv = jnp.zeros((16,), jnp.float32)
    for e in range(16):
        for j in range(C // 16):
            dr_v[e, pl.ds(j * 16, 16)] = zv

    def body(i, carry):
        g = i * NW + wid

        @pl.when(g < NG)
        def _():
            pltpu.sync_copy(logits_hbm.at[g], lg_v)
            pltpu.sync_copy(dst_hbm.at[pl.ds(g * 16, 16)], dst_v)
            for h in range(H):
                mh = plsc.load_gather(m_v, [_splat(h)])
                ex_v[h] = jnp.exp(lg_v[h] - mh)
            for e in range(16):
                row = plsc.load_gather(ex_v, [hsel, _splat(e)])
                dr_v[e, pl.ds(0, 16)] = jnp.where(hmask, row, 0.0)
            pltpu.sync_copy(ex_v, ex_out.at[g])
            pltpu.sync_copy(dr_v, den_sh.at[dst_v], add=True)

        return carry

    lax.fori_loop(0, GPW, body, 0)
    plsc.subcore_barrier()
    pltpu.sync_copy(
        den_sh.at[pl.ds(sid * RPS, RPS)],
        den_out.at[pl.ds(cid * NP + sid * RPS, RPS)],
    )


@functools.partial(
    pl.kernel,
    out_type=jax.ShapeDtypeStruct((2 * NP, C), jnp.float32),
    mesh=_mesh,
    compiler_params=pltpu.CompilerParams(needs_layout_passes=False),
    scratch_types=[
        pltpu.VMEM_SHARED((NP, C), jnp.float32),
        pltpu.VMEM((16, HC), jnp.float32),
        pltpu.VMEM((16, C), jnp.float32),
        pltpu.VMEM((8, 16), jnp.float32),
        pltpu.VMEM((8, 16), jnp.float32),
        pltpu.VMEM((16, C), jnp.float32),
        pltpu.VMEM((16,), jnp.int32),
        pltpu.VMEM((16,), jnp.int32),
        pltpu.SemaphoreType.DMA,
        pltpu.SemaphoreType.DMA,
    ],
)
def _k3_aggregate(ex_hbm, rden_hbm, xl_hbm, src_hbm, dst_hbm, zrows_hbm,
                  acc_out, acc_sh, xlr, rdr, ex_v, wv_v, ct_v, src_v, dst_v,
                  sem1, sem2):
    cid = lax.axis_index("c")
    sid = lax.axis_index("s")
    wid = _wid()
    pltpu.sync_copy(zrows_hbm.at[pl.ds(0, RPS)], acc_sh.at[pl.ds(sid * RPS, RPS)])
    plsc.subcore_barrier()
    lanes = _lanes()

    def body(i, carry):
        g = i * NW + wid

        @pl.when(g < NG)
        def _():
            base = g * 16
            pltpu.sync_copy(src_hbm.at[pl.ds(base, 16)], src_v)
            pltpu.sync_copy(dst_hbm.at[pl.ds(base, 16)], dst_v)
            pltpu.sync_copy(ex_hbm.at[g], ex_v)
            cp1 = pltpu.async_copy(xl_hbm.at[src_v], xlr, sem1)
            cp2 = pltpu.async_copy(rden_hbm.at[dst_v], rdr, sem2)
            cp1.wait()
            cp2.wait()
            for h in range(H):
                rdh = plsc.load_gather(rdr, [lanes, _splat(h)])
                wv_v[h] = ex_v[h] * rdh
            for e in range(16):
                whe = [
                    plsc.load_gather(wv_v, [_splat(h), _splat(e)])
                    for h in range(H)
                ]
                for j in range(C // 16):
                    cv = jnp.zeros((16,), jnp.float32)
                    for h in range(H):
                        cv = cv + whe[h] * xlr[e, pl.ds(h * C + j * 16, 16)]
                    ct_v[e, pl.ds(j * 16, 16)] = cv
            pltpu.sync_copy(ct_v, acc_sh.at[dst_v], add=True)

        return carry

    lax.fori_loop(0, GPW, body, 0)
    plsc.subcore_barrier()
    pltpu.sync_copy(
        acc_sh.at[pl.ds(sid * RPS, RPS)],
        acc_out.at[pl.ds(cid * NP + sid * RPS, RPS)],
    )


def _p0_body(h0_ref, h1_ref, rext_ref, out_ref):
    hist = h0_ref[...] + h1_ref[...]
    ea = jnp.dot(hist, rext_ref[...], preferred_element_type=jnp.float32)
    cnt = ea[:, D:D + 1]
    out_ref[...] = ea[:, :D] / jnp.maximum(cnt, 1.0)


def _p0_loop_attr(h0, h1, rext):
    return pl.pallas_call(
        _p0_body,
        grid=(NP // NBLK,),
        in_specs=[
            pl.BlockSpec((NBLK, C), lambda i: (i, 0)),
            pl.BlockSpec((NBLK, C), lambda i: (i, 0)),
            pl.BlockSpec((2 * R, 2 * D), lambda i: (0, 0)),
        ],
        out_specs=pl.BlockSpec((NBLK, D), lambda i: (i, 0)),
        out_shape=jax.ShapeDtypeStruct((NP, D), jnp.float32),
    )(h0, h1, rext)


def _pre_body(rel_ref, we_ref, be_ref, out_ref):
    out_ref[...] = (
        jnp.dot(rel_ref[...], we_ref[...], preferred_element_type=jnp.float32)
        + be_ref[...]
    )


def _p_re(relations, we, be_row):
    return pl.pallas_call(
        _pre_body,
        in_specs=[
            pl.BlockSpec((R, D), lambda: (0, 0)),
            pl.BlockSpec((D, HC), lambda: (0, 0)),
            pl.BlockSpec((1, HC), lambda: (0, 0)),
        ],
        out_specs=pl.BlockSpec((R, HC), lambda: (0, 0)),
        out_shape=jax.ShapeDtypeStruct((R, HC), jnp.float32),
    )(relations, we, be_row)


def _p1_body(h_ref, la_ref, w2_ref, b2_ref, we_ref, be_ref, att_ref,
             xl_ref, xr_ref, llog_ref):
    proj = (
        jnp.dot(h_ref[...], w2_ref[...], preferred_element_type=jnp.float32)
        + b2_ref[...]
    )
    le = (
        jnp.dot(la_ref[...], we_ref[...], preferred_element_type=jnp.float32)
        + be_ref[...]
    )
    xl = proj[:, :HC]
    xr = proj[:, HC:]
    xl_ref[...] = xl
    xr_ref[...] = xr
    zz = xl + xr + le
    zz = jnp.maximum(zz, NEG * zz) * att_ref[...]
    cols = []
    for h in range(H):
        cols.append(jnp.sum(zz[:, h * C:(h + 1) * C], axis=1, keepdims=True))
    llog = jnp.concatenate(
        cols + [jnp.full((NBLK, 16 - H), -1e30, jnp.float32)], axis=1
    )
    llog_ref[...] = llog


def _p1_project(h_pad, la, w2, b2, we, be_row, att_row):
    return pl.pallas_call(
        _p1_body,
        grid=(NP // NBLK,),
        in_specs=[
            pl.BlockSpec((NBLK, D), lambda i: (i, 0)),
            pl.BlockSpec((NBLK, D), lambda i: (i, 0)),
            pl.BlockSpec((D, 2 * HC), lambda i: (0, 0)),
            pl.BlockSpec((1, 2 * HC), lambda i: (0, 0)),
            pl.BlockSpec((D, HC), lambda i: (0, 0)),
            pl.BlockSpec((1, HC), lambda i: (0, 0)),
            pl.BlockSpec((1, HC), lambda i: (0, 0)),
        ],
        out_specs=[
            pl.BlockSpec((NBLK, HC), lambda i: (i, 0)),
            pl.BlockSpec((NBLK, HC), lambda i: (i, 0)),
            pl.BlockSpec((NBLK, 16), lambda i: (i, 0)),
        ],
        out_shape=[
            jax.ShapeDtypeStruct((NP, HC), jnp.float32),
            jax.ShapeDtypeStruct((NP, HC), jnp.float32),
            jax.ShapeDtypeStruct((NP, 16), jnp.float32),
        ],
    )(h_pad, la, w2, b2, we, be_row, att_row)


def _p3_body(maxw_ref, llog_ref, out_ref):
    m = jnp.max(maxw_ref[...], axis=0, keepdims=True)
    m = jnp.maximum(m, jnp.max(llog_ref[...], axis=0, keepdims=True))
    out_ref[...] = m


def _p3_max(maxw, llog):
    return pl.pallas_call(
        _p3_body,
        in_specs=[
            pl.BlockSpec((NW, 16), lambda: (0, 0)),
            pl.BlockSpec((NP, 16), lambda: (0, 0)),
        ],
        out_specs=pl.BlockSpec((1, 16), lambda: (0, 0)),
        out_shape=jax.ShapeDtypeStruct((1, 16), jnp.float32),
    )(maxw, llog)


def _p4_body(d0_ref, d1_ref, llog_ref, m_ref, out_ref):
    lex = jnp.exp(llog_ref[...] - m_ref[...])
    den = d0_ref[...][:, :16] + d1_ref[...][:, :16] + lex
    rden = 1.0 / (den + 1e-16)
    out_ref[...] = jnp.concatenate(
        [rden, jnp.zeros((NBLK, C - 16), jnp.float32)], axis=1
    )


def _p4_rden(d0, d1, llog, m):
    return pl.pallas_call(
        _p4_body,
        grid=(NP // NBLK,),
        in_specs=[
            pl.BlockSpec((NBLK, C), lambda i: (i, 0)),
            pl.BlockSpec((NBLK, C), lambda i: (i, 0)),
            pl.BlockSpec((NBLK, 16), lambda i: (i, 0)),
            pl.BlockSpec((1, 16), lambda i: (0, 0)),
        ],
        out_specs=pl.BlockSpec((NBLK, C), lambda i: (i, 0)),
        out_shape=jax.ShapeDtypeStruct((NP, C), jnp.float32),
    )(d0, d1, llog, m)


def _p5_body(a0_ref, a1_ref, llog_ref, rden_ref, m_ref, xl_ref, bias_ref,
             out_ref):
    lw = jnp.exp(llog_ref[...] - m_ref[...]) * rden_ref[...][:, :16]
    xl = xl_ref[...]
    lp = jnp.zeros((NBLK, C), jnp.float32)
    for h in range(H):
        lp = lp + lw[:, h:h + 1] * xl[:, h * C:(h + 1) * C]
    out_ref[...] = (a0_ref[...] + a1_ref[...] + lp) * (1.0 / H) + bias_ref[...]


def _p5_combine(a0, a1, llog, rden, m, xl, bias_row):
    return pl.pallas_call(
        _p5_body,
        grid=(NP // NBLK,),
        in_specs=[
            pl.BlockSpec((NBLK, C), lambda i: (i, 0)),
            pl.BlockSpec((NBLK, C), lambda i: (i, 0)),
            pl.BlockSpec((NBLK, 16), lambda i: (i, 0)),
            pl.BlockSpec((NBLK, C), lambda i: (i, 0)),
            pl.BlockSpec((1, 16), lambda i: (0, 0)),
            pl.BlockSpec((NBLK, HC), lambda i: (i, 0)),
            pl.BlockSpec((1, C), lambda i: (0, 0)),
        ],
        out_specs=pl.BlockSpec((NBLK, C), lambda i: (i, 0)),
        out_shape=jax.ShapeDtypeStruct((NP, C), jnp.float32),
    )(a0, a1, llog, rden, m, xl, bias_row)


def kernel(x, edge_index, relations, relation_index, Wl, bl, Wr, br, We, be,
           att, bias):
    src = edge_index[0]
    dst = edge_index[1]
    rel = relation_index

    zrows128 = jnp.zeros((RPS, C), jnp.float32)

    hist2 = _k0_hist(dst, rel, zrows128)
    rext = jnp.concatenate(
        [
            jnp.concatenate(
                [relations, jnp.ones((R, 1), jnp.float32),
                 jnp.zeros((R, D - 1), jnp.float32)],
                axis=1,
            ),
            jnp.zeros((R, 2 * D), jnp.float32),
        ],
        axis=0,
    )
    la = _p0_loop_attr(hist2[:NP], hist2[NP:], rext)

    h_pad = jnp.pad(x, ((0, NP - N), (0, 0)))
    for l in range(L):
        w2 = jnp.concatenate([Wl[l], Wr[l]], axis=1)
        b2 = jnp.concatenate([bl[l], br[l]])[None, :]
        be_row = be[l][None, :]
        att_row = att[l].reshape(1, HC)

        re2 = _p_re(relations, We[l], be_row)
        xl, xr, llog = _p1_project(h_pad, la, w2, b2, We[l], be_row, att_row)
        logits, maxw = _k1_logits(xl, xr, re2, att[l].reshape(HC), src, dst, rel)
        m = _p3_max(maxw, llog)
        ex, den = _k2_softmax(logits, m.reshape(16), dst, zrows128)
        rden = _p4_rden(den[:NP], den[NP:], llog, m)
        acc = _k3_aggregate(ex, rden, xl, src, dst, zrows128)
        h_pad = _p5_combine(acc[:NP], acc[NP:], llog, rden, m, xl, bias[l][None, :])

    return (h_pad[:N], relations)


# speedup vs baseline: 1.1039x; 1.1039x over previous
import functools

import jax
import jax.numpy as jnp
from jax import lax
from jax.experimental import pallas as pl
from jax.experimental.pallas import tpu as pltpu
from jax.experimental.pallas import tpu_sc as plsc

N = 10000
E = 160000
D = 128
H = 8
C = 128
L = 4
R = 64
NEG = 0.2
HC = H * C

NP = 10240
NBLK = 1024
NC, NS = 2, 16
NW = NC * NS
NG = E // 16
GPW = (NG + NW - 1) // NW
IPW = GPW * 16
GCH = 64
EPC = GCH * 16
RPS = NP // NS

_mesh = plsc.VectorSubcoreMesh(core_axis_name="c", subcore_axis_name="s")


def _wid():
    return lax.axis_index("s") * NC + lax.axis_index("c")


def _lanes():
    return lax.iota(jnp.int32, 16)


def _splat(v):
    return jnp.full((16,), v, jnp.int32)


@functools.partial(
    pl.kernel,
    out_type=jax.ShapeDtypeStruct((2 * NP, C), jnp.float32),
    mesh=_mesh,
    compiler_params=pltpu.CompilerParams(needs_layout_passes=False),
    scratch_types=[
        pltpu.VMEM_SHARED((NP, C), jnp.float32),
        pltpu.VMEM((16, C), jnp.float32),
        pltpu.VMEM((16,), jnp.int32),
        pltpu.VMEM((16,), jnp.int32),
    ],
)
def _k0_hist(dst_hbm, rel_hbm, zrows_hbm, hist_out, hist_sh, ones_v, dst_v, rel_v):
    cid = lax.axis_index("c")
    sid = lax.axis_index("s")
    wid = _wid()
    pltpu.sync_copy(zrows_hbm, hist_sh.at[pl.ds(sid * RPS, RPS)])
    plsc.subcore_barrier()
    lanes = _lanes()

    def body(i, carry):
        g = i * NW + wid

        @pl.when(g < NG)
        def _():
            pltpu.sync_copy(dst_hbm.at[pl.ds(g * 16, 16)], dst_v)
            pltpu.sync_copy(rel_hbm.at[pl.ds(g * 16, 16)], rel_v)
            for e in range(16):
                rel_e = plsc.load_gather(rel_v, [_splat(e)])
                for j in range(C // 16):
                    ones_v[e, pl.ds(j * 16, 16)] = jnp.where(
                        j * 16 + lanes == rel_e, 1.0, 0.0
                    )
            pltpu.sync_copy(ones_v, hist_sh.at[dst_v], add=True)

        return carry

    lax.fori_loop(0, GPW, body, 0)
    plsc.subcore_barrier()
    pltpu.sync_copy(
        hist_sh.at[pl.ds(sid * RPS, RPS)],
        hist_out.at[pl.ds(cid * NP + sid * RPS, RPS)],
    )


@functools.partial(
    pl.kernel,
    out_type=(
        jax.ShapeDtypeStruct((NG, 8, 16), jnp.float32),
        jax.ShapeDtypeStruct((NW, 16), jnp.float32),
    ),
    mesh=_mesh,
    compiler_params=pltpu.CompilerParams(needs_layout_passes=False),
    scratch_types=[
        pltpu.VMEM((EPC,), jnp.int32),
        pltpu.VMEM((EPC,), jnp.int32),
        pltpu.VMEM((EPC,), jnp.int32),
        pltpu.VMEM((HC,), jnp.float32),
        pltpu.VMEM((16, HC), jnp.float32),
        pltpu.VMEM((16, HC), jnp.float32),
        pltpu.VMEM((16, HC), jnp.float32),
        pltpu.VMEM((16, HC), jnp.float32),
        pltpu.VMEM((16, HC), jnp.float32),
        pltpu.VMEM((16, HC), jnp.float32),
        pltpu.VMEM((8, 16), jnp.float32),
        pltpu.VMEM((16,), jnp.float32),
        pltpu.SemaphoreType.DMA,
        pltpu.SemaphoreType.DMA,
        pltpu.SemaphoreType.DMA,
        pltpu.SemaphoreType.DMA,
        pltpu.SemaphoreType.DMA,
        pltpu.SemaphoreType.DMA,
    ],
)
def _k1_logits(xl_hbm, xr_hbm, re2_hbm, att_hbm, src_hbm, dst_hbm, rel_hbm,
               logits_out, maxw_out,
               srcv, dstv, relv_all, att_v, xla, xlb, xra, xrb, rea, reb,
               lg_v, mxc_v, sla, slb, sra, srb, sea, seb):
    wid = _wid()
    base = wid * GPW - jnp.maximum(wid - NS, 0)
    n_g = GPW - (wid >= NS).astype(jnp.int32)
    pltpu.sync_copy(att_hbm, att_v)
    lanes = _lanes()
    mxc_v[...] = jnp.full((16,), -1e30, jnp.float32)

    def _issue(i, xlbuf, xrbuf, rebuf, s1, s2, s3):
        @pl.when(i < n_g)
        def _():
            loc = i % GCH

            @pl.when(loc == 0)
            def _():
                pltpu.sync_copy(src_hbm.at[pl.ds((base + i) * 16, EPC)], srcv)
                pltpu.sync_copy(dst_hbm.at[pl.ds((base + i) * 16, EPC)], dstv)
                pltpu.sync_copy(rel_hbm.at[pl.ds((base + i) * 16, EPC)], relv_all)

            pltpu.async_copy(xl_hbm.at[srcv.at[pl.ds(loc * 16, 16)]], xlbuf, s1)
            pltpu.async_copy(xr_hbm.at[dstv.at[pl.ds(loc * 16, 16)]], xrbuf, s2)
            pltpu.async_copy(re2_hbm.at[relv_all.at[pl.ds(loc * 16, 16)]], rebuf, s3)

    def _wait(i, xlbuf, xrbuf, rebuf, s1, s2, s3):
        @pl.when(i < n_g)
        def _():
            pltpu.make_async_copy(xl_hbm.at[pl.ds(0, 16)], xlbuf, s1).wait()
            pltpu.make_async_copy(xr_hbm.at[pl.ds(0, 16)], xrbuf, s2).wait()
            pltpu.make_async_copy(re2_hbm.at[pl.ds(0, 16)], rebuf, s3).wait()

    def _compute(i, xlbuf, xrbuf, rebuf):
        @pl.when(i < n_g)
        def _():
            g = base + i
            mxc = mxc_v[...]
            for h in range(H):
                def fbody(c, acc):
                    fsp = _splat(h * C + c)
                    xlv = plsc.load_gather(xlbuf, [lanes, fsp])
                    xrv = plsc.load_gather(xrbuf, [lanes, fsp])
                    rev = plsc.load_gather(rebuf, [lanes, fsp])
                    attv = plsc.load_gather(att_v, [fsp])
                    z = xlv + xrv + rev
                    z = jnp.maximum(z, NEG * z)
                    return acc + z * attv

                logit = lax.fori_loop(0, C, fbody, jnp.zeros((16,), jnp.float32))
                lg_v[h] = logit
                mh = jnp.max(logit)
                mxc = jnp.where(lanes == h, jnp.maximum(mxc, mh), mxc)
            mxc_v[...] = mxc
            pltpu.sync_copy(lg_v, logits_out.at[g])

    _issue(0, xla, xra, rea, sla, sra, sea)

    def body2(k, carry):
        i0 = 2 * k
        _wait(i0, xla, xra, rea, sla, sra, sea)
        _issue(i0 + 1, xlb, xrb, reb, slb, srb, seb)
        _compute(i0, xla, xra, rea)
        _wait(i0 + 1, xlb, xrb, reb, slb, srb, seb)
        _issue(i0 + 2, xla, xra, rea, sla, sra, sea)
        _compute(i0 + 1, xlb, xrb, reb)
        return carry

    lax.fori_loop(0, (GPW + 1) // 2, body2, 0)
    pltpu.sync_copy(mxc_v, maxw_out.at[wid])


@functools.partial(
    pl.kernel,
    out_type=(
        jax.ShapeDtypeStruct((NG, 8, 16), jnp.float32),
        jax.ShapeDtypeStruct((2 * NP, C), jnp.float32),
    ),
    mesh=_mesh,
    compiler_params=pltpu.CompilerParams(needs_layout_passes=False),
    scratch_types=[
        pltpu.VMEM_SHARED((NP, C), jnp.float32),
        pltpu.VMEM((16,), jnp.float32),
        pltpu.VMEM((8, 16), jnp.float32),
        pltpu.VMEM((8, 16), jnp.float32),
        pltpu.VMEM((16, C), jnp.float32),
        pltpu.VMEM((16,), jnp.int32),
    ],
)
def _k2_softmax(logits_hbm, m_hbm, dst_hbm, zrows_hbm, ex_out, den_out,
                den_sh, m_v, lg_v, ex_v, dr_v, dst_v):
    cid = lax.axis_index("c")
    sid = lax.axis_index("s")
    wid = _wid()
    pltpu.sync_copy(m_hbm, m_v)
    pltpu.sync_copy(zrows_hbm, den_sh.at[pl.ds(sid * RPS, RPS)])
    plsc.subcore_barrier()
    lanes = _lanes()
    hsel = jnp.minimum(lanes, H - 1)
    hmask = lanes < H
    zv = jnp.zeros((16,), jnp.float32)
    for e in range(16):
        for j in range(C // 16):
            dr_v[e, pl.ds(j * 16, 16)] = zv

    def body(i, carry):
        g = i * NW + wid

        @pl.when(g < NG)
        def _():
            pltpu.sync_copy(logits_hbm.at[g], lg_v)
            pltpu.sync_copy(dst_hbm.at[pl.ds(g * 16, 16)], dst_v)
            for h in range(H):
                mh = plsc.load_gather(m_v, [_splat(h)])
                ex_v[h] = jnp.exp(lg_v[h] - mh)
            for e in range(16):
                row = plsc.load_gather(ex_v, [hsel, _splat(e)])
                dr_v[e, pl.ds(0, 16)] = jnp.where(hmask, row, 0.0)
            pltpu.sync_copy(ex_v, ex_out.at[g])
            pltpu.sync_copy(dr_v, den_sh.at[dst_v], add=True)

        return carry

    lax.fori_loop(0, GPW, body, 0)
    plsc.subcore_barrier()
    pltpu.sync_copy(
        den_sh.at[pl.ds(sid * RPS, RPS)],
        den_out.at[pl.ds(cid * NP + sid * RPS, RPS)],
    )


@functools.partial(
    pl.kernel,
    out_type=jax.ShapeDtypeStruct((2 * NP, C), jnp.float32),
    mesh=_mesh,
    compiler_params=pltpu.CompilerParams(needs_layout_passes=False),
    scratch_types=[
        pltpu.VMEM_SHARED((NP, C), jnp.float32),
        pltpu.VMEM((EPC,), jnp.int32),
        pltpu.VMEM((EPC,), jnp.int32),
        pltpu.VMEM((16, HC), jnp.float32),
        pltpu.VMEM((16, HC), jnp.float32),
        pltpu.VMEM((16, C), jnp.float32),
        pltpu.VMEM((16, C), jnp.float32),
        pltpu.VMEM((8, 16), jnp.float32),
        pltpu.VMEM((8, 16), jnp.float32),
        pltpu.VMEM((16, C), jnp.float32),
        pltpu.VMEM((16,), jnp.int32),
        pltpu.SemaphoreType.DMA,
        pltpu.SemaphoreType.DMA,
        pltpu.SemaphoreType.DMA,
        pltpu.SemaphoreType.DMA,
    ],
)
def _k3_aggregate(ex_hbm, rden_hbm, xl_hbm, src_hbm, dst_hbm, zrows_hbm,
                  acc_out, acc_sh, srcv, dstv, xla, xlb, rda, rdb,
                  ex_v, wv_v, ct_v, dst_v, sxa, sxb, sra, srb):
    cid = lax.axis_index("c")
    sid = lax.axis_index("s")
    wid = _wid()
    base = wid * GPW - jnp.maximum(wid - NS, 0)
    n_g = GPW - (wid >= NS).astype(jnp.int32)
    pltpu.sync_copy(zrows_hbm.at[pl.ds(0, RPS)], acc_sh.at[pl.ds(sid * RPS, RPS)])
    plsc.subcore_barrier()
    lanes = _lanes()

    def _issue(i, xlbuf, rdbuf, s1, s2):
        @pl.when(i < n_g)
        def _():
            loc = i % GCH

            @pl.when(loc == 0)
            def _():
                pltpu.sync_copy(src_hbm.at[pl.ds((base + i) * 16, EPC)], srcv)
                pltpu.sync_copy(dst_hbm.at[pl.ds((base + i) * 16, EPC)], dstv)

            pltpu.async_copy(xl_hbm.at[srcv.at[pl.ds(loc * 16, 16)]], xlbuf, s1)
            pltpu.async_copy(rden_hbm.at[dstv.at[pl.ds(loc * 16, 16)]], rdbuf, s2)

    def _wait(i, xlbuf, rdbuf, s1, s2):
        @pl.when(i < n_g)
        def _():
            pltpu.make_async_copy(xl_hbm.at[pl.ds(0, 16)], xlbuf, s1).wait()
            pltpu.make_async_copy(rden_hbm.at[pl.ds(0, 16)], rdbuf, s2).wait()

    def _compute(i, xlbuf, rdbuf):
        @pl.when(i < n_g)
        def _():
            g = base + i
            pltpu.sync_copy(ex_hbm.at[g], ex_v)
            pltpu.sync_copy(dst_hbm.at[pl.ds(base * 16 + i * 16, 16)], dst_v)
            for h in range(H):
                rdh = plsc.load_gather(rdbuf, [lanes, _splat(h)])
                wv_v[h] = ex_v[h] * rdh
            for e in range(16):
                whe = [
                    plsc.load_gather(wv_v, [_splat(h), _splat(e)])
                    for h in range(H)
                ]
                for j in range(C // 16):
                    cv = jnp.zeros((16,), jnp.float32)
                    for h in range(H):
                        cv = cv + whe[h] * xlbuf[e, pl.ds(h * C + j * 16, 16)]
                    ct_v[e, pl.ds(j * 16, 16)] = cv
            pltpu.sync_copy(ct_v, acc_sh.at[dst_v], add=True)

    _issue(0, xla, rda, sxa, sra)

    def body2(k, carry):
        i0 = 2 * k
        _wait(i0, xla, rda, sxa, sra)
        _issue(i0 + 1, xlb, rdb, sxb, srb)
        _compute(i0, xla, rda)
        _wait(i0 + 1, xlb, rdb, sxb, srb)
        _issue(i0 + 2, xla, rda, sxa, sra)
        _compute(i0 + 1, xlb, rdb)
        return carry

    lax.fori_loop(0, (GPW + 1) // 2, body2, 0)
    plsc.subcore_barrier()
    pltpu.sync_copy(
        acc_sh.at[pl.ds(sid * RPS, RPS)],
        acc_out.at[pl.ds(cid * NP + sid * RPS, RPS)],
    )


def _p0_body(h0_ref, h1_ref, rext_ref, out_ref):
    hist = h0_ref[...] + h1_ref[...]
    ea = jnp.dot(hist, rext_ref[...], preferred_element_type=jnp.float32)
    cnt = ea[:, D:D + 1]
    out_ref[...] = ea[:, :D] / jnp.maximum(cnt, 1.0)


def _p0_loop_attr(h0, h1, rext):
    return pl.pallas_call(
        _p0_body,
        grid=(NP // NBLK,),
        in_specs=[
            pl.BlockSpec((NBLK, C), lambda i: (i, 0)),
            pl.BlockSpec((NBLK, C), lambda i: (i, 0)),
            pl.BlockSpec((2 * R, 2 * D), lambda i: (0, 0)),
        ],
        out_specs=pl.BlockSpec((NBLK, D), lambda i: (i, 0)),
        out_shape=jax.ShapeDtypeStruct((NP, D), jnp.float32),
    )(h0, h1, rext)


def _pre_body(rel_ref, we_ref, be_ref, out_ref):
    out_ref[...] = (
        jnp.dot(rel_ref[...], we_ref[...], preferred_element_type=jnp.float32)
        + be_ref[...]
    )


def _p_re(relations, we, be_row):
    return pl.pallas_call(
        _pre_body,
        in_specs=[
            pl.BlockSpec((R, D), lambda: (0, 0)),
            pl.BlockSpec((D, HC), lambda: (0, 0)),
            pl.BlockSpec((1, HC), lambda: (0, 0)),
        ],
        out_specs=pl.BlockSpec((R, HC), lambda: (0, 0)),
        out_shape=jax.ShapeDtypeStruct((R, HC), jnp.float32),
    )(relations, we, be_row)


def _p1_body(h_ref, la_ref, w2_ref, b2_ref, we_ref, be_ref, att_ref,
             xl_ref, xr_ref, llog_ref):
    proj = (
        jnp.dot(h_ref[...], w2_ref[...], preferred_element_type=jnp.float32)
        + b2_ref[...]
    )
    le = (
        jnp.dot(la_ref[...], we_ref[...], preferred_element_type=jnp.float32)
        + be_ref[...]
    )
    xl = proj[:, :HC]
    xr = proj[:, HC:]
    xl_ref[...] = xl
    xr_ref[...] = xr
    zz = xl + xr + le
    zz = jnp.maximum(zz, NEG * zz) * att_ref[...]
    cols = []
    for h in range(H):
        cols.append(jnp.sum(zz[:, h * C:(h + 1) * C], axis=1, keepdims=True))
    llog = jnp.concatenate(
        cols + [jnp.full((NBLK, 16 - H), -1e30, jnp.float32)], axis=1
    )
    llog_ref[...] = llog


def _p1_project(h_pad, la, w2, b2, we, be_row, att_row):
    return pl.pallas_call(
        _p1_body,
        grid=(NP // NBLK,),
        in_specs=[
            pl.BlockSpec((NBLK, D), lambda i: (i, 0)),
            pl.BlockSpec((NBLK, D), lambda i: (i, 0)),
            pl.BlockSpec((D, 2 * HC), lambda i: (0, 0)),
            pl.BlockSpec((1, 2 * HC), lambda i: (0, 0)),
            pl.BlockSpec((D, HC), lambda i: (0, 0)),
            pl.BlockSpec((1, HC), lambda i: (0, 0)),
            pl.BlockSpec((1, HC), lambda i: (0, 0)),
        ],
        out_specs=[
            pl.BlockSpec((NBLK, HC), lambda i: (i, 0)),
            pl.BlockSpec((NBLK, HC), lambda i: (i, 0)),
            pl.BlockSpec((NBLK, 16), lambda i: (i, 0)),
        ],
        out_shape=[
            jax.ShapeDtypeStruct((NP, HC), jnp.float32),
            jax.ShapeDtypeStruct((NP, HC), jnp.float32),
            jax.ShapeDtypeStruct((NP, 16), jnp.float32),
        ],
    )(h_pad, la, w2, b2, we, be_row, att_row)


def _p3_body(maxw_ref, llog_ref, out_ref):
    m = jnp.max(maxw_ref[...], axis=0, keepdims=True)
    m = jnp.maximum(m, jnp.max(llog_ref[...], axis=0, keepdims=True))
    out_ref[...] = m


def _p3_max(maxw, llog):
    return pl.pallas_call(
        _p3_body,
        in_specs=[
            pl.BlockSpec((NW, 16), lambda: (0, 0)),
            pl.BlockSpec((NP, 16), lambda: (0, 0)),
        ],
        out_specs=pl.BlockSpec((1, 16), lambda: (0, 0)),
        out_shape=jax.ShapeDtypeStruct((1, 16), jnp.float32),
    )(maxw, llog)


def _p4_body(d0_ref, d1_ref, llog_ref, m_ref, out_ref):
    lex = jnp.exp(llog_ref[...] - m_ref[...])
    den = d0_ref[...][:, :16] + d1_ref[...][:, :16] + lex
    rden = 1.0 / (den + 1e-16)
    out_ref[...] = jnp.concatenate(
        [rden, jnp.zeros((NBLK, C - 16), jnp.float32)], axis=1
    )


def _p4_rden(d0, d1, llog, m):
    return pl.pallas_call(
        _p4_body,
        grid=(NP // NBLK,),
        in_specs=[
            pl.BlockSpec((NBLK, C), lambda i: (i, 0)),
            pl.BlockSpec((NBLK, C), lambda i: (i, 0)),
            pl.BlockSpec((NBLK, 16), lambda i: (i, 0)),
            pl.BlockSpec((1, 16), lambda i: (0, 0)),
        ],
        out_specs=pl.BlockSpec((NBLK, C), lambda i: (i, 0)),
        out_shape=jax.ShapeDtypeStruct((NP, C), jnp.float32),
    )(d0, d1, llog, m)


def _p5_body(a0_ref, a1_ref, llog_ref, rden_ref, m_ref, xl_ref, bias_ref,
             out_ref):
    lw = jnp.exp(llog_ref[...] - m_ref[...]) * rden_ref[...][:, :16]
    xl = xl_ref[...]
    lp = jnp.zeros((NBLK, C), jnp.float32)
    for h in range(H):
        lp = lp + lw[:, h:h + 1] * xl[:, h * C:(h + 1) * C]
    out_ref[...] = (a0_ref[...] + a1_ref[...] + lp) * (1.0 / H) + bias_ref[...]


def _p5_combine(a0, a1, llog, rden, m, xl, bias_row):
    return pl.pallas_call(
        _p5_body,
        grid=(NP // NBLK,),
        in_specs=[
            pl.BlockSpec((NBLK, C), lambda i: (i, 0)),
            pl.BlockSpec((NBLK, C), lambda i: (i, 0)),
            pl.BlockSpec((NBLK, 16), lambda i: (i, 0)),
            pl.BlockSpec((NBLK, C), lambda i: (i, 0)),
            pl.BlockSpec((1, 16), lambda i: (0, 0)),
            pl.BlockSpec((NBLK, HC), lambda i: (i, 0)),
            pl.BlockSpec((1, C), lambda i: (0, 0)),
        ],
        out_specs=pl.BlockSpec((NBLK, C), lambda i: (i, 0)),
        out_shape=jax.ShapeDtypeStruct((NP, C), jnp.float32),
    )(a0, a1, llog, rden, m, xl, bias_row)


def kernel(x, edge_index, relations, relation_index, Wl, bl, Wr, br, We, be,
           att, bias):
    src = jnp.pad(edge_index[0], (0, EPC + 32))
    dst = jnp.pad(edge_index[1], (0, EPC + 32))
    rel = jnp.pad(relation_index, (0, EPC + 32))

    zrows128 = jnp.zeros((RPS, C), jnp.float32)

    hist2 = _k0_hist(dst, rel, zrows128)
    rext = jnp.concatenate(
        [
            jnp.concatenate(
                [relations, jnp.ones((R, 1), jnp.float32),
                 jnp.zeros((R, D - 1), jnp.float32)],
                axis=1,
            ),
            jnp.zeros((R, 2 * D), jnp.float32),
        ],
        axis=0,
    )
    la = _p0_loop_attr(hist2[:NP], hist2[NP:], rext)

    h_pad = jnp.pad(x, ((0, NP - N), (0, 0)))
    for l in range(L):
        w2 = jnp.concatenate([Wl[l], Wr[l]], axis=1)
        b2 = jnp.concatenate([bl[l], br[l]])[None, :]
        be_row = be[l][None, :]
        att_row = att[l].reshape(1, HC)

        re2 = _p_re(relations, We[l], be_row)
        xl, xr, llog = _p1_project(h_pad, la, w2, b2, We[l], be_row, att_row)
        logits, maxw = _k1_logits(xl, xr, re2, att[l].reshape(HC), src, dst, rel)
        m = _p3_max(maxw, llog)
        ex, den = _k2_softmax(logits, m.reshape(16), dst, zrows128)
        rden = _p4_rden(den[:NP], den[NP:], llog, m)
        acc = _k3_aggregate(ex, rden, xl, src, dst, zrows128)
        h_pad = _p5_combine(acc[:NP], acc[NP:], llog, rden, m, xl, bias[l][None, :])

    return (h_pad[:N], relations)
